# baseline (speedup 1.0000x reference)
import functools

import jax
import jax.numpy as jnp
from jax import lax
from jax.experimental import pallas as pl
from jax.experimental.pallas import tpu as pltpu
from jax.experimental.pallas import tpu_sc as plsc

_MAX_BOXES = 20
_SCORE_THRESHOLD = 0.3
_IOU_THRESHOLD = 0.1
_NUM_CLASSES = 80
_N_BOXES = 20000
_L = 16
_NWORKERS = 32
_BIG = 2**30
_OUT_PAD = 32


def _sc_nms_body(scores_hbm, boxes_hbm, out_hbm, b_ref, s_ref, sidx_ref, obuf):
    wid = lax.axis_index("s") * 2 + lax.axis_index("c")
    pltpu.sync_copy(boxes_hbm, b_ref)

    iota16 = lax.iota(jnp.int32, _L)
    zeros16 = jnp.zeros((_L,), jnp.int32)
    neg_inf = jnp.float32(-jnp.inf)
    neginf16 = jnp.full((_L,), neg_inf, jnp.float32)
    big16 = jnp.full((_L,), _BIG, jnp.int32)

    def process_class(c):
        pltpu.sync_copy(scores_hbm.at[c], s_ref)

        def init_chunk(i, carry):
            w, bv, bi = carry
            base = i * _L
            sv = s_ref[pl.ds(base, _L)]
            idxv = base + iota16
            mask = sv >= _SCORE_THRESHOLD
            plsc.store_compressed(sidx_ref.at[pl.ds(w, _L)], idxv, mask=mask)
            cnt = jnp.max(plsc.all_reduce_population_count(mask))
            v = jnp.where(mask, sv, neg_inf)
            better = (v > bv) | ((v == bv) & (idxv < bi))
            bv = jnp.where(better, v, bv)
            bi = jnp.where(better, idxv, bi)
            return (w + cnt, bv, bi)

        w, bestv, besti = lax.fori_loop(
            0, _N_BOXES // _L, init_chunk, (jnp.int32(0), neginf16, big16)
        )
        sidx_ref[pl.ds(w, _L)] = zeros16

        def iter_body(t, icarry):
            n, bestv, besti, out0, out1 = icarry
            m = jnp.max(bestv)
            ii = jnp.where(bestv == m, besti, big16)
            sel = jnp.min(ii)
            ok = n > 0
            outval16 = jnp.broadcast_to(jnp.where(ok, sel, -1), (_L,))
            out0 = jnp.where(iota16 == t, outval16, out0)
            out1 = jnp.where(iota16 == t - _L, outval16, out1)
            sel_safe = jnp.where(ok, sel, 0)
            sel16 = jnp.broadcast_to(sel_safe, (_L,))

            sy1 = plsc.load_gather(b_ref, [zeros16, sel16])
            sx1 = plsc.load_gather(b_ref, [zeros16 + 1, sel16])
            sy2 = plsc.load_gather(b_ref, [zeros16 + 2, sel16])
            sx2 = plsc.load_gather(b_ref, [zeros16 + 3, sel16])
            area_a = (sy2 - sy1) * (sx2 - sx1)

            def sweep_chunk(i, carry):
                w, bv, bi = carry
                base = i * _L
                idxv = sidx_ref[pl.ds(base, _L)]
                lanemask = (base + iota16) < n
                sv = plsc.load_gather(s_ref, [idxv])
                by1 = plsc.load_gather(b_ref, [zeros16, idxv])
                bx1 = plsc.load_gather(b_ref, [zeros16 + 1, idxv])
                by2 = plsc.load_gather(b_ref, [zeros16 + 2, idxv])
                bx2 = plsc.load_gather(b_ref, [zeros16 + 3, idxv])
                iy1 = jnp.maximum(sy1, by1)
                ix1 = jnp.maximum(sx1, bx1)
                iy2 = jnp.minimum(sy2, by2)
                ix2 = jnp.minimum(sx2, bx2)
                inter = jnp.maximum(0.0, iy2 - iy1) * jnp.maximum(0.0, ix2 - ix1)
                area_b = (by2 - by1) * (bx2 - bx1)
                iou = inter / (area_a + area_b - inter + jnp.float32(1e-9))
                keep = (
                    lanemask
                    & jnp.logical_not(iou > _IOU_THRESHOLD)
                    & (idxv != sel16)
                )
                plsc.store_compressed(sidx_ref.at[pl.ds(w, _L)], idxv, mask=keep)
                cnt = jnp.max(plsc.all_reduce_population_count(keep))
                v = jnp.where(keep, sv, neg_inf)
                better = (v > bv) | ((v == bv) & (idxv < bi))
                bv = jnp.where(better, v, bv)
                bi = jnp.where(better, idxv, bi)
                return (w + cnt, bv, bi)

            nch = (n + _L - 1) // _L
            w2, bv2, bi2 = lax.fori_loop(
                0, nch, sweep_chunk, (jnp.int32(0), neginf16, big16)
            )
            sidx_ref[pl.ds(w2, _L)] = zeros16
            return (w2, bv2, bi2, out0, out1)

        out_init = jnp.full((_L,), -1, jnp.int32)
        _, _, _, out0, out1 = lax.fori_loop(
            0, _MAX_BOXES, iter_body, (w, bestv, besti, out_init, out_init)
        )

        obuf[pl.ds(0, _L)] = out0
        obuf[pl.ds(_L, _L)] = out1
        pltpu.sync_copy(obuf, out_hbm.at[c])

    def class_body(r, _):
        c = wid + _NWORKERS * r

        @pl.when(c < _NUM_CLASSES)
        def _():
            process_class(c)

        return 0

    lax.fori_loop(0, 3, class_body, 0)


@functools.partial(
    pl.kernel,
    out_type=jax.ShapeDtypeStruct((_NUM_CLASSES, _OUT_PAD), jnp.int32),
    compiler_params=pltpu.CompilerParams(needs_layout_passes=False),
    mesh=plsc.VectorSubcoreMesh(core_axis_name="c", subcore_axis_name="s"),
    scratch_types=[
        pltpu.VMEM((4, _N_BOXES), jnp.float32),
        pltpu.VMEM((_N_BOXES,), jnp.float32),
        pltpu.VMEM((_N_BOXES + _L,), jnp.int32),
        pltpu.VMEM((_OUT_PAD,), jnp.int32),
    ],
)
def _sc_nms(scores_hbm, boxes_hbm, out_hbm, b_ref, s_ref, sidx_ref, obuf):
    _sc_nms_body(scores_hbm, boxes_hbm, out_hbm, b_ref, s_ref, sidx_ref, obuf)


@jax.jit
def kernel(boxes, box_scores):
    scores_t = box_scores.T
    boxes_t = boxes.T

    nms_idx = _sc_nms(scores_t, boxes_t)[:, :_MAX_BOXES]

    classes = jnp.broadcast_to(
        jnp.arange(_NUM_CLASSES, dtype=jnp.int32)[:, None], nms_idx.shape
    )
    batch = jnp.zeros_like(nms_idx)
    valid = (nms_idx >= 0).reshape(-1, 1)
    nms_final = jnp.stack([batch, classes, nms_idx], axis=-1).reshape(-1, 3)
    nms_final = jnp.where(valid, nms_final, -1)
    return boxes[None], scores_t[None], nms_final[None]


# speedup vs baseline: 3.5651x; 3.5651x over previous
import functools

import jax
import jax.numpy as jnp
from jax import lax
from jax.experimental import pallas as pl
from jax.experimental.pallas import tpu as pltpu
from jax.experimental.pallas import tpu_sc as plsc

_MAX_BOXES = 20
_SCORE_THRESHOLD = 0.3
_IOU_THRESHOLD = 0.1
_NUM_CLASSES = 80
_N_BOXES = 20000
_L = 16
_NWORKERS = 32
_BIG = 2**30
_OUT_PAD = 32


def _sc_nms_body(scores_hbm, boxes_hbm, out_hbm, b_ref, s_ref, area_ref, obuf):
    wid = lax.axis_index("s") * 2 + lax.axis_index("c")
    pltpu.sync_copy(boxes_hbm, b_ref)

    iota16 = lax.iota(jnp.int32, _L)
    neg_inf = jnp.float32(-jnp.inf)
    neginf16 = jnp.full((_L,), neg_inf, jnp.float32)
    big16 = jnp.full((_L,), _BIG, jnp.int32)

    @plsc.parallel_loop(0, _N_BOXES, _L, unroll=4)
    def _area_chunk(i):
        by1 = b_ref[0, pl.ds(i, _L)]
        bx1 = b_ref[1, pl.ds(i, _L)]
        by2 = b_ref[2, pl.ds(i, _L)]
        bx2 = b_ref[3, pl.ds(i, _L)]
        area_ref[pl.ds(i, _L)] = (by2 - by1) * (bx2 - bx1)

    def process_class(c):
        pltpu.sync_copy(scores_hbm.at[c], s_ref)

        @plsc.parallel_loop(0, _N_BOXES, _L, unroll=4, carry=(neginf16, big16))
        def _thresh(i, carry):
            bv, bi = carry
            sv = s_ref[pl.ds(i, _L)]
            v = jnp.where(sv >= _SCORE_THRESHOLD, sv, neg_inf)
            s_ref[pl.ds(i, _L)] = v
            idxv = i + iota16
            better = (v > bv) | ((v == bv) & (idxv < bi))
            return (jnp.where(better, v, bv), jnp.where(better, idxv, bi))

        bestv0, besti0 = _thresh

        def iter_body(t, icarry):
            bestv, besti, out0, out1 = icarry
            m = jnp.max(bestv)
            ii = jnp.where(bestv == m, besti, big16)
            sel = jnp.min(ii)
            ok = m > neg_inf
            outval16 = jnp.broadcast_to(jnp.where(ok, sel, -1), (_L,))
            out0 = jnp.where(iota16 == t, outval16, out0)
            out1 = jnp.where(iota16 == t - _L, outval16, out1)
            sel16 = jnp.broadcast_to(jnp.where(ok, sel, 0), (_L,))
            okv = jnp.broadcast_to(ok, (_L,))

            sy1 = plsc.load_gather(b_ref, [jnp.zeros((_L,), jnp.int32), sel16])
            sx1 = plsc.load_gather(b_ref, [jnp.full((_L,), 1, jnp.int32), sel16])
            sy2 = plsc.load_gather(b_ref, [jnp.full((_L,), 2, jnp.int32), sel16])
            sx2 = plsc.load_gather(b_ref, [jnp.full((_L,), 3, jnp.int32), sel16])
            area_a = (sy2 - sy1) * (sx2 - sx1)

            @plsc.parallel_loop(0, _N_BOXES, _L, unroll=4, carry=(neginf16, big16))
            def _sweep(i, carry):
                bv, bi = carry
                sv = s_ref[pl.ds(i, _L)]
                by1 = b_ref[0, pl.ds(i, _L)]
                bx1 = b_ref[1, pl.ds(i, _L)]
                by2 = b_ref[2, pl.ds(i, _L)]
                bx2 = b_ref[3, pl.ds(i, _L)]
                ab = area_ref[pl.ds(i, _L)]
                iy1 = jnp.maximum(sy1, by1)
                ix1 = jnp.maximum(sx1, bx1)
                iy2 = jnp.minimum(sy2, by2)
                ix2 = jnp.minimum(sx2, bx2)
                inter = jnp.maximum(0.0, iy2 - iy1) * jnp.maximum(0.0, ix2 - ix1)
                iou = inter / (area_a + ab - inter + jnp.float32(1e-9))
                idxv = i + iota16
                kill = ((iou > _IOU_THRESHOLD) & okv) | (idxv == sel16)
                v = jnp.where(kill, neg_inf, sv)
                s_ref[pl.ds(i, _L)] = v
                better = (v > bv) | ((v == bv) & (idxv < bi))
                return (jnp.where(better, v, bv), jnp.where(better, idxv, bi))

            bestv, besti = _sweep
            return (bestv, besti, out0, out1)

        out_init = jnp.full((_L,), -1, jnp.int32)
        _, _, out0, out1 = lax.fori_loop(
            0, _MAX_BOXES, iter_body, (bestv0, besti0, out_init, out_init)
        )

        obuf[pl.ds(0, _L)] = out0
        obuf[pl.ds(_L, _L)] = out1
        pltpu.sync_copy(obuf, out_hbm.at[c])

    def class_body(r, _):
        c = wid + _NWORKERS * r

        @pl.when(c < _NUM_CLASSES)
        def _():
            process_class(c)

        return 0

    lax.fori_loop(0, 3, class_body, 0)


@functools.partial(
    pl.kernel,
    out_type=jax.ShapeDtypeStruct((_NUM_CLASSES, _OUT_PAD), jnp.int32),
    compiler_params=pltpu.CompilerParams(needs_layout_passes=False),
    mesh=plsc.VectorSubcoreMesh(core_axis_name="c", subcore_axis_name="s"),
    scratch_types=[
        pltpu.VMEM((4, _N_BOXES), jnp.float32),
        pltpu.VMEM((_N_BOXES,), jnp.float32),
        pltpu.VMEM((_N_BOXES,), jnp.float32),
        pltpu.VMEM((_OUT_PAD,), jnp.int32),
    ],
)
def _sc_nms(scores_hbm, boxes_hbm, out_hbm, b_ref, s_ref, area_ref, obuf):
    _sc_nms_body(scores_hbm, boxes_hbm, out_hbm, b_ref, s_ref, area_ref, obuf)


@jax.jit
def kernel(boxes, box_scores):
    scores_t = box_scores.T
    boxes_t = boxes.T

    nms_idx = _sc_nms(scores_t, boxes_t)[:, :_MAX_BOXES]

    classes = jnp.broadcast_to(
        jnp.arange(_NUM_CLASSES, dtype=jnp.int32)[:, None], nms_idx.shape
    )
    batch = jnp.zeros_like(nms_idx)
    valid = (nms_idx >= 0).reshape(-1, 1)
    nms_final = jnp.stack([batch, classes, nms_idx], axis=-1).reshape(-1, 3)
    nms_final = jnp.where(valid, nms_final, -1)
    return boxes[None], scores_t[None], nms_final[None]
